# baseline (speedup 1.0000x reference)
import functools

import jax
import jax.numpy as jnp
from jax import lax
from jax.experimental import pallas as pl
from jax.experimental.pallas import tpu as pltpu
from jax.experimental.pallas import tpu_sc as plsc

N = 32768
D = 256
NC = 2
NS = 16
NW = NC * NS
RPW = N // NW
CH = 64
NCH = RPW // CH
NBUF = 3
NTB = 3
L = 16
V = 17
G = CH // L
GR = 16
GG = GR // L


def _sc_body(x_hbm, idx_hbm, idxg_hbm, tab_hbm, out_hbm,
             tab_v, idx_v, idxg_v, xb0, xb1, xb2, tb0, tb1, tb2,
             xsems, gsems, ssems):
    wid = lax.axis_index("s") * NC + lax.axis_index("c")
    base = wid * RPW
    xbufs = [xb0, xb1, xb2]
    tbufs = [tb0, tb1, tb2]

    pltpu.sync_copy(tab_hbm, tab_v)
    pltpu.sync_copy(idx_hbm.at[wid], idx_v)
    pltpu.sync_copy(idxg_hbm.at[wid], idxg_v)
    for ci in range(NCH):
        for j in range(GR // L):
            sl = (ci, pl.ds(j * L, L))
            idxg_v[sl] = jnp.clip(idxg_v[sl], 0, 16)

    def load(ci, b):
        cx = pltpu.async_copy(
            x_hbm.at[pl.ds(base + ci * CH, CH)], xbufs[b], xsems.at[b])
        cg = pltpu.async_copy(
            tab_hbm.at[idxg_v.at[ci]],
            tbufs[ci % NTB], gsems.at[ci % NTB])
        return (cx, cg)

    def store(ci, b):
        return pltpu.async_copy(
            xbufs[b], out_hbm.at[pl.ds(base + ci * CH, CH)], ssems.at[b])

    rowvecs = [g * L + lax.iota(jnp.int32, L) for g in range(G)]
    iotav = lax.iota(jnp.int32, L)

    loads, stores = {}, {}
    for k in range(min(NBUF - 1, NCH)):
        loads[k] = load(k, k % NBUF)

    for ci in range(NCH):
        b = ci % NBUF
        k = ci + NBUF - 1
        if k < NCH:
            if k >= NBUF:
                stores.pop(k - NBUF).wait()
            loads[k] = load(k, k % NBUF)
        cx, cg = loads.pop(ci)
        cx.wait()

        xb = xbufs[b]
        tb = tbufs[ci % NTB]
        ivecs = [jnp.clip(idx_v[ci, pl.ds(g * L, L)], 0, 16)
                 for g in range(GG, G)]

        @plsc.parallel_loop(0, D, 1, unroll=2)
        def _(t):
            tv = jnp.full((L,), t, jnp.int32)
            cvec = (tv & ~15) + ((iotav + tv) & 15)
            for g in range(G - GG):
                tval = plsc.load_gather(tab_v, [ivecs[g], cvec])
                plsc.addupdate_scatter(xb, [rowvecs[GG + g], cvec], tval)

        cg.wait()

        def row_add(r, _):
            for j in range(D // L):
                plsc.addupdate(xb.at[r, pl.ds(j * L, L)],
                               tb[r, pl.ds(j * L, L)])
            return 0

        lax.fori_loop(0, GR, row_add, 0)
        stores[ci] = store(ci, b)
    for ci in sorted(stores):
        stores.pop(ci).wait()


@jax.jit
def _sc_call(xr, idx3, table):
    mesh = plsc.VectorSubcoreMesh(core_axis_name="c", subcore_axis_name="s")
    f = functools.partial(
        pl.kernel,
        mesh=mesh,
        compiler_params=pltpu.CompilerParams(
            use_tc_tiling_on_sc=False, needs_layout_passes=False),
        out_type=jax.ShapeDtypeStruct((N, D), jnp.float32),
        scratch_types=[
            pltpu.VMEM((V, D), jnp.float32),
            pltpu.VMEM((NCH, CH), jnp.int32),
            pltpu.VMEM((NCH, GR), jnp.int32),
            pltpu.VMEM((CH, D), jnp.float32),
            pltpu.VMEM((CH, D), jnp.float32),
            pltpu.VMEM((CH, D), jnp.float32),
            pltpu.VMEM((GR, D), jnp.float32),
            pltpu.VMEM((GR, D), jnp.float32),
            pltpu.VMEM((GR, D), jnp.float32),
            pltpu.SemaphoreType.DMA((NBUF,)),
            pltpu.SemaphoreType.DMA((NTB,)),
            pltpu.SemaphoreType.DMA((NBUF,)),
        ],
    )(_sc_body)
    return f(xr, idx3, idx3[:, :, :GR], table)


def kernel(x, in_chan_matrix, embed_table):
    B, S, Dd = x.shape
    xr = x.reshape(B * S, Dd)
    idx3 = in_chan_matrix.astype(jnp.int32).reshape(NW, NCH, CH)
    out = _sc_call(xr, idx3, embed_table)
    return out.reshape(B, S, Dd)
